# baseline (speedup 1.0000x reference)
import functools

import jax
import jax.numpy as jnp
from jax import lax
from jax.experimental import pallas as pl
from jax.experimental.pallas import tpu as pltpu
from jax.experimental.pallas import tpu_sc as plsc

N = 10000
E = 320000
DF = 128
H = 1024
A = 64
G = 16

NC, NS, L = 2, 16, 16
NW = NC * NS
NP = 10240
RT = NP // NS
K = 112
CH = 90
EP = NW * CH * K
NB = 3
ZC = 80
KD = K
CHD = EP // (NW * KD)
TRASH = N

def _make_deg(mesh):
    @functools.partial(
        pl.kernel,
        out_type=jax.ShapeDtypeStruct((2 * NP, L), jnp.float32),
        mesh=mesh,
        scratch_types=[
            pltpu.VMEM((CHD, KD), jnp.int32),
            pltpu.VMEM((NP,), jnp.float32),
            pltpu.VMEM((NS, RT), jnp.float32),
            pltpu.VMEM((RT, L), jnp.float32),
            pltpu.VMEM_SHARED((NS, NP), jnp.float32),
            pltpu.SemaphoreType.DMA,
        ],
        compiler_params=pltpu.CompilerParams(needs_layout_passes=False),
    )
    def _deg_sc(dst_hbm, out_hbm, dst_v, part_v, red_v, rep_v, deg_sh, sem):
        c = lax.axis_index("c")
        s = lax.axis_index("s")
        wid = c * NS + s
        zero = jnp.zeros((L,), jnp.float32)
        ones = jnp.ones((L,), jnp.float32)

        def _z(i, carry):
            part_v[pl.ds(i * L, L)] = zero
            return carry

        lax.fori_loop(0, NP // L, _z, 0)

        pltpu.sync_copy(dst_hbm.at[wid], dst_v)

        def _scat(ch, carry):
            for j in range(KD // L):
                idx = dst_v[ch, pl.ds(j * L, L)]
                plsc.addupdate_scatter(part_v, [idx], ones)
            return carry

        lax.fori_loop(0, CHD, _scat, 0)
        pltpu.sync_copy(part_v, deg_sh.at[s])
        plsc.subcore_barrier()

        pltpu.sync_copy(deg_sh.at[:, pl.ds(s * RT, RT)], red_v)

        def _red(g, carry):
            tot = red_v[0, pl.ds(g * L, L)]
            for t in range(1, NS):
                tot = tot + red_v[t, pl.ds(g * L, L)]
            part_v[pl.ds(g * L, L)] = tot
            return carry

        lax.fori_loop(0, RT // L, _red, 0)

        def _rep(g, carry):
            vals = part_v[pl.ds(g * L, L)]
            for t in range(L):
                rep_v[g * L + t] = ones * vals[t]
            return carry

        lax.fori_loop(0, RT // L, _rep, 0)
        pltpu.sync_copy(
            rep_v, out_hbm.at[pl.ds(c * NP + s * RT, RT)]
        )

    return _deg_sc


NST = 6
CHH = CH // NST


def _make_agg(mesh, D):
    @functools.partial(
        pl.kernel,
        out_type=jax.ShapeDtypeStruct((2 * NP, D), jnp.float32),
        mesh=mesh,
        scratch_types=[
            pltpu.VMEM((CHH, K), jnp.int32),
            pltpu.VMEM((CHH, K), jnp.int32),
            pltpu.VMEM((K, D), jnp.float32),
            pltpu.VMEM((K, D), jnp.float32),
            pltpu.VMEM((K, D), jnp.float32),
            pltpu.VMEM_SHARED((NP, D), jnp.float32),
            pltpu.SemaphoreType.DMA,
            pltpu.SemaphoreType.DMA,
            pltpu.SemaphoreType.DMA,
            pltpu.SemaphoreType.DMA,
            pltpu.SemaphoreType.DMA,
            pltpu.SemaphoreType.DMA,
        ],
    )
    def _agg(u_hbm, src_hbm, dst_hbm, out_hbm, src_v, dst_v,
             r0, r1, r2, acc_sh, g0, g1, g2, s0, s1, s2):
        rows = (r0, r1, r2)
        gsem = (g0, g1, g2)
        ssem = (s0, s1, s2)
        c = lax.axis_index("c")
        s = lax.axis_index("s")
        wid = c * NS + s
        zero = jnp.zeros((L,), jnp.float32)

        def _zrow(i, carry):
            for j in range(D // L):
                r0[i, pl.ds(j * L, L)] = zero
            return carry

        lax.fori_loop(0, K, _zrow, 0)

        def _zcp(j, carry):
            pltpu.sync_copy(
                r0.at[pl.ds(0, ZC)], acc_sh.at[pl.ds(s * RT + j * ZC, ZC)]
            )
            return carry

        lax.fori_loop(0, RT // ZC, _zcp, 0)
        plsc.subcore_barrier()

        for half in range(NST):
            pltpu.sync_copy(src_hbm.at[wid, half], src_v)
            pltpu.sync_copy(dst_hbm.at[wid, half], dst_v)
            gd = [
                pltpu.async_copy(u_hbm.at[src_v.at[p]], rows[p], gsem[p])
                for p in range(NB)
            ]

            def _edge(i, carry):
                base = i * NB
                for p in range(NB):
                    ch = base + p
                    gd[p].wait()
                    pltpu.async_copy(
                        rows[p], acc_sh.at[dst_v.at[ch]], ssem[p], add=True
                    ).wait()

                    @pl.when(ch + NB < CHH)
                    def _():
                        pltpu.async_copy(
                            u_hbm.at[src_v.at[ch + NB]], rows[p], gsem[p]
                        )

                return carry

            lax.fori_loop(0, CHH // NB, _edge, 0)

        plsc.subcore_barrier()
        pltpu.sync_copy(
            acc_sh.at[pl.ds(s * RT, RT)], out_hbm.at[pl.ds(c * NP + s * RT, RT)]
        )

    return _agg


@functools.lru_cache(maxsize=1)
def _sc_kernels():
    mesh = plsc.VectorSubcoreMesh(
        core_axis_name="c", subcore_axis_name="s", num_cores=NC, num_subcores=NS
    )
    return _make_deg(mesh), _make_agg(mesh, DF)


def _prep_body(degp, xp, w2, wl, b2, bl, dis_o, u1_o, wc_o, bc_o):
    d = degp[...]
    deg = d[0:NP, 0:1] + d[NP : 2 * NP, 0:1] + 1.0
    dis = lax.rsqrt(deg)
    dis_o[...] = dis
    u1_o[...] = xp[...] * dis
    wc = jnp.dot(w2[...], wl[...], preferred_element_type=jnp.float32)
    wc_o[...] = jnp.pad(wc, ((0, 0), (0, DF - A)))
    bc_o[...] = (
        jnp.dot(b2[...], wl[...], preferred_element_type=jnp.float32) + bl[...]
    )


_prep = pl.pallas_call(
    _prep_body,
    out_shape=[
        jax.ShapeDtypeStruct((NP, 1), jnp.float32),
        jax.ShapeDtypeStruct((NP, DF), jnp.float32),
        jax.ShapeDtypeStruct((H, DF), jnp.float32),
        jax.ShapeDtypeStruct((1, A), jnp.float32),
    ],
)

RB = 1024
GR = NP // RB


def _mid_body(a0, a1, u1, dis, w1, b1, wc, u2_o):
    z1 = dis[...] * (a0[...] + a1[...] + u1[...])
    h1 = jnp.maximum(
        jnp.dot(z1, w1[...], preferred_element_type=jnp.float32) + b1[...], 0.0
    )
    u2_o[...] = dis[...] * jnp.dot(h1, wc[...], preferred_element_type=jnp.float32)


_mid = pl.pallas_call(
    _mid_body,
    grid=(GR,),
    in_specs=[
        pl.BlockSpec((RB, DF), lambda i: (i, 0)),
        pl.BlockSpec((RB, DF), lambda i: (i + GR, 0)),
        pl.BlockSpec((RB, DF), lambda i: (i, 0)),
        pl.BlockSpec((RB, 1), lambda i: (i, 0)),
        pl.BlockSpec((DF, H), lambda i: (0, 0)),
        pl.BlockSpec((1, H), lambda i: (0, 0)),
        pl.BlockSpec((H, DF), lambda i: (0, 0)),
    ],
    out_specs=pl.BlockSpec((RB, DF), lambda i: (i, 0)),
    out_shape=jax.ShapeDtypeStruct((NP, DF), jnp.float32),
)


def _final_body(a2, u2, dis, bc, batch2, out):
    acc = a2[...]
    z2 = dis[...] * (
        acc[0:NP, 0:A] + acc[NP : 2 * NP, 0:A] + u2[...][:, 0:A]
    )
    y = jnp.tanh(z2 + bc[...])
    gids = lax.broadcasted_iota(jnp.int32, (G, NP), 0)
    p = (batch2[...] == gids).astype(jnp.float32)
    sums = jnp.dot(p, y, preferred_element_type=jnp.float32)
    counts = jnp.sum(p, axis=1, keepdims=True)
    out[...] = sums / jnp.maximum(counts, 1.0)


_final = pl.pallas_call(
    _final_body,
    out_shape=jax.ShapeDtypeStruct((G, A), jnp.float32),
)


def kernel(x, edge_index, batch, W1, b1, W2, b2, Wl, bl):
    src = edge_index[0].astype(jnp.int32)
    dst = edge_index[1].astype(jnp.int32)
    pad = TRASH + jnp.arange(EP - E, dtype=jnp.int32) % (NP - N)
    src3 = jnp.concatenate([src, pad]).reshape(NW, NST, CHH, K)
    dst3 = jnp.concatenate([dst, pad]).reshape(NW, NST, CHH, K)
    dstd = dst3.reshape(NW, CHD, KD)
    xp = jnp.pad(x, ((0, NP - N), (0, 0)))
    batch2 = jnp.pad(
        batch.astype(jnp.int32), (0, NP - N), constant_values=G
    ).reshape(1, NP)

    deg_sc, agg128 = _sc_kernels()
    degp = deg_sc(dstd)
    dis, u1, wc, bc = _prep(degp, xp, W2, Wl, b2.reshape(1, H), bl.reshape(1, A))
    acc1 = agg128(u1, src3, dst3)
    u2 = _mid(acc1, acc1, u1, dis, W1, b1.reshape(1, H), wc)
    acc2 = agg128(u2, src3, dst3)
    return _final(acc2, u2, dis, bc, batch2)


# speedup vs baseline: 1.0695x; 1.0695x over previous
import functools

import jax
import jax.numpy as jnp
from jax import lax
from jax.experimental import pallas as pl
from jax.experimental.pallas import tpu as pltpu
from jax.experimental.pallas import tpu_sc as plsc

N = 10000
E = 320000
DF = 128
H = 1024
A = 64
G = 16

NC, NS, L = 2, 16, 16
NW = NC * NS
NP = 10240
RT = NP // NS
K = 96
CH = 108
EP = NW * CH * K
NB = 3
ZC = 80
KD = 128
CHD = EP // (NW * KD)
TRASH = N

def _make_deg(mesh):
    @functools.partial(
        pl.kernel,
        out_type=jax.ShapeDtypeStruct((2 * NP, L), jnp.float32),
        mesh=mesh,
        scratch_types=[
            pltpu.VMEM((CHD, KD), jnp.int32),
            pltpu.VMEM((NP,), jnp.float32),
            pltpu.VMEM((NS, RT), jnp.float32),
            pltpu.VMEM((RT, L), jnp.float32),
            pltpu.VMEM_SHARED((NS, NP), jnp.float32),
            pltpu.SemaphoreType.DMA,
        ],
        compiler_params=pltpu.CompilerParams(needs_layout_passes=False),
    )
    def _deg_sc(dst_hbm, out_hbm, dst_v, part_v, red_v, rep_v, deg_sh, sem):
        c = lax.axis_index("c")
        s = lax.axis_index("s")
        wid = c * NS + s
        zero = jnp.zeros((L,), jnp.float32)
        ones = jnp.ones((L,), jnp.float32)

        def _z(i, carry):
            part_v[pl.ds(i * L, L)] = zero
            return carry

        lax.fori_loop(0, NP // L, _z, 0)

        pltpu.sync_copy(dst_hbm.at[wid], dst_v)

        def _scat(ch, carry):
            for j in range(KD // L):
                idx = dst_v[ch, pl.ds(j * L, L)]
                plsc.addupdate_scatter(part_v, [idx], ones)
            return carry

        lax.fori_loop(0, CHD, _scat, 0)
        pltpu.sync_copy(part_v, deg_sh.at[s])
        plsc.subcore_barrier()

        pltpu.sync_copy(deg_sh.at[:, pl.ds(s * RT, RT)], red_v)

        def _red(g, carry):
            tot = red_v[0, pl.ds(g * L, L)]
            for t in range(1, NS):
                tot = tot + red_v[t, pl.ds(g * L, L)]
            part_v[pl.ds(g * L, L)] = tot
            return carry

        lax.fori_loop(0, RT // L, _red, 0)

        def _rep(g, carry):
            vals = part_v[pl.ds(g * L, L)]
            for t in range(L):
                rep_v[g * L + t] = ones * vals[t]
            return carry

        lax.fori_loop(0, RT // L, _rep, 0)
        pltpu.sync_copy(
            rep_v, out_hbm.at[pl.ds(c * NP + s * RT, RT)]
        )

    return _deg_sc


NST = 3
CHH = CH // NST


def _make_agg(mesh, D):
    @functools.partial(
        pl.kernel,
        out_type=jax.ShapeDtypeStruct((2 * NP, D), jnp.float32),
        mesh=mesh,
        scratch_types=[
            pltpu.VMEM((CHH, K), jnp.int32),
            pltpu.VMEM((CHH, K), jnp.int32),
            pltpu.VMEM((K, D), jnp.float32),
            pltpu.VMEM((K, D), jnp.float32),
            pltpu.VMEM((K, D), jnp.float32),
            pltpu.VMEM_SHARED((NP, D), jnp.float32),
            pltpu.SemaphoreType.DMA,
            pltpu.SemaphoreType.DMA,
            pltpu.SemaphoreType.DMA,
            pltpu.SemaphoreType.DMA,
            pltpu.SemaphoreType.DMA,
            pltpu.SemaphoreType.DMA,
        ],
    )
    def _agg(u_hbm, src_hbm, dst_hbm, out_hbm, src_v, dst_v,
             r0, r1, r2, acc_sh, g0, g1, g2, s0, s1, s2):
        rows = (r0, r1, r2)
        gsem = (g0, g1, g2)
        ssem = (s0, s1, s2)
        c = lax.axis_index("c")
        s = lax.axis_index("s")
        wid = c * NS + s
        zero = jnp.zeros((L,), jnp.float32)

        def _zrow(i, carry):
            for j in range(D // L):
                r0[i, pl.ds(j * L, L)] = zero
            return carry

        lax.fori_loop(0, K, _zrow, 0)

        def _zcp(j, carry):
            pltpu.sync_copy(
                r0.at[pl.ds(0, ZC)], acc_sh.at[pl.ds(s * RT + j * ZC, ZC)]
            )
            return carry

        lax.fori_loop(0, RT // ZC, _zcp, 0)
        plsc.subcore_barrier()

        for half in range(NST):
            pltpu.sync_copy(src_hbm.at[wid, half], src_v)
            pltpu.sync_copy(dst_hbm.at[wid, half], dst_v)
            gd = [
                pltpu.async_copy(u_hbm.at[src_v.at[p]], rows[p], gsem[p])
                for p in range(NB)
            ]

            def _edge(i, carry):
                base = i * NB
                for p in range(NB):
                    ch = base + p
                    gd[p].wait()
                    pltpu.async_copy(
                        rows[p], acc_sh.at[dst_v.at[ch]], ssem[p], add=True
                    ).wait()

                    @pl.when(ch + NB < CHH)
                    def _():
                        pltpu.async_copy(
                            u_hbm.at[src_v.at[ch + NB]], rows[p], gsem[p]
                        )

                return carry

            lax.fori_loop(0, CHH // NB, _edge, 0)

        plsc.subcore_barrier()
        pltpu.sync_copy(
            acc_sh.at[pl.ds(s * RT, RT)], out_hbm.at[pl.ds(c * NP + s * RT, RT)]
        )

    return _agg


@functools.lru_cache(maxsize=1)
def _sc_kernels():
    mesh = plsc.VectorSubcoreMesh(
        core_axis_name="c", subcore_axis_name="s", num_cores=NC, num_subcores=NS
    )
    return _make_deg(mesh), _make_agg(mesh, DF)


def _prep_body(degp, xp, w2, wl, b2, bl, dis_o, u1_o, wc_o, bc_o):
    d = degp[...]
    deg = d[0:NP, 0:1] + d[NP : 2 * NP, 0:1] + 1.0
    dis = lax.rsqrt(deg)
    dis_o[...] = dis
    u1_o[...] = xp[...] * dis
    wc = jnp.dot(w2[...], wl[...], preferred_element_type=jnp.float32)
    wc_o[...] = jnp.pad(wc, ((0, 0), (0, DF - A)))
    bc_o[...] = (
        jnp.dot(b2[...], wl[...], preferred_element_type=jnp.float32) + bl[...]
    )


_prep = pl.pallas_call(
    _prep_body,
    out_shape=[
        jax.ShapeDtypeStruct((NP, 1), jnp.float32),
        jax.ShapeDtypeStruct((NP, DF), jnp.float32),
        jax.ShapeDtypeStruct((H, DF), jnp.float32),
        jax.ShapeDtypeStruct((1, A), jnp.float32),
    ],
)

RB = 1024
GR = NP // RB


def _mid_body(a0, a1, u1, dis, w1, b1, wc, u2_o):
    z1 = dis[...] * (a0[...] + a1[...] + u1[...])
    h1 = jnp.maximum(
        jnp.dot(z1, w1[...], preferred_element_type=jnp.float32) + b1[...], 0.0
    )
    u2_o[...] = dis[...] * jnp.dot(h1, wc[...], preferred_element_type=jnp.float32)


_mid = pl.pallas_call(
    _mid_body,
    grid=(GR,),
    in_specs=[
        pl.BlockSpec((RB, DF), lambda i: (i, 0)),
        pl.BlockSpec((RB, DF), lambda i: (i + GR, 0)),
        pl.BlockSpec((RB, DF), lambda i: (i, 0)),
        pl.BlockSpec((RB, 1), lambda i: (i, 0)),
        pl.BlockSpec((DF, H), lambda i: (0, 0)),
        pl.BlockSpec((1, H), lambda i: (0, 0)),
        pl.BlockSpec((H, DF), lambda i: (0, 0)),
    ],
    out_specs=pl.BlockSpec((RB, DF), lambda i: (i, 0)),
    out_shape=jax.ShapeDtypeStruct((NP, DF), jnp.float32),
)


def _final_body(a2, u2, dis, bc, batch2, out):
    acc = a2[...]
    z2 = dis[...] * (
        acc[0:NP, 0:A] + acc[NP : 2 * NP, 0:A] + u2[...][:, 0:A]
    )
    y = jnp.tanh(z2 + bc[...])
    gids = lax.broadcasted_iota(jnp.int32, (G, NP), 0)
    p = (batch2[...] == gids).astype(jnp.float32)
    sums = jnp.dot(p, y, preferred_element_type=jnp.float32)
    counts = jnp.sum(p, axis=1, keepdims=True)
    out[...] = sums / jnp.maximum(counts, 1.0)


_final = pl.pallas_call(
    _final_body,
    out_shape=jax.ShapeDtypeStruct((G, A), jnp.float32),
)


def kernel(x, edge_index, batch, W1, b1, W2, b2, Wl, bl):
    src = edge_index[0].astype(jnp.int32)
    dst = edge_index[1].astype(jnp.int32)
    pad = TRASH + jnp.arange(EP - E, dtype=jnp.int32) % (NP - N)
    src3 = jnp.concatenate([src, pad]).reshape(NW, NST, CHH, K)
    dst3 = jnp.concatenate([dst, pad]).reshape(NW, NST, CHH, K)
    dstd = dst3.reshape(NW, CHD, KD)
    xp = jnp.pad(x, ((0, NP - N), (0, 0)))
    batch2 = jnp.pad(
        batch.astype(jnp.int32), (0, NP - N), constant_values=G
    ).reshape(1, NP)

    deg_sc, agg128 = _sc_kernels()
    degp = deg_sc(dstd)
    dis, u1, wc, bc = _prep(degp, xp, W2, Wl, b2.reshape(1, H), bl.reshape(1, A))
    acc1 = agg128(u1, src3, dst3)
    u2 = _mid(acc1, acc1, u1, dis, W1, b1.reshape(1, H), wc)
    acc2 = agg128(u2, src3, dst3)
    return _final(acc2, u2, dis, bc, batch2)
